# baseline (speedup 1.0000x reference)
import functools

import jax
import jax.numpy as jnp
from jax import lax
from jax.experimental import pallas as pl
from jax.experimental.pallas import tpu as pltpu
from jax.experimental.pallas import tpu_sc as plsc

E = 8
K = 2
H = 1024
I = 2048
T = 2048
EPAD = 128

B = 512
NT = 15
P = NT * B
BI = 512
NC_I = I // BI

NC = 2
NS = 16
NW = NC * NS
CHUNK = T // NW
SUB = 32
GW = 128


def _meta_body(x_ref, rw_ref, posc_ref, wexp_ref, texp_ref, tval_ref):
    x = x_ref[...]
    logits = lax.dot_general(
        x, rw_ref[...], (((1,), (1,)), ((), ())),
        preferred_element_type=jnp.float32,
    )
    lane = lax.broadcasted_iota(jnp.int32, (T, EPAD), 1)
    valid = lane < E
    logits = jnp.where(valid, logits, -1e30)
    m = jnp.max(logits, axis=1, keepdims=True)
    p = jnp.where(valid, jnp.exp(logits - m), 0.0)
    probs = p / jnp.sum(p, axis=1, keepdims=True)

    m1 = jnp.max(probs, axis=1, keepdims=True)
    idx1 = jnp.min(jnp.where(probs == m1, lane, EPAD), axis=1, keepdims=True)
    probs2 = jnp.where(lane == idx1, -1.0, probs)
    m2 = jnp.max(probs2, axis=1, keepdims=True)
    idx2 = jnp.min(jnp.where(probs2 == m2, lane, EPAD), axis=1, keepdims=True)

    oh0 = (lane == idx1).astype(jnp.bfloat16)
    oh1 = (lane == idx2).astype(jnp.bfloat16)

    r_i = lax.broadcasted_iota(jnp.int32, (T, T), 0)
    c_i = lax.broadcasted_iota(jnp.int32, (T, T), 1)
    s_tri = (r_i > c_i).astype(jnp.bfloat16)
    rank0 = lax.dot_general(
        s_tri, oh0, (((1,), (0,)), ((), ())),
        preferred_element_type=jnp.float32,
    )
    cnt0 = jnp.sum(oh0.astype(jnp.float32), axis=0, keepdims=True)
    rank1 = lax.dot_general(
        s_tri, oh1, (((1,), (0,)), ((), ())),
        preferred_element_type=jnp.float32,
    ) + cnt0
    cnt1 = jnp.sum(oh1.astype(jnp.float32), axis=0, keepdims=True)
    counts = (cnt0 + cnt1).astype(jnp.int32)

    pc = ((counts + (B - 1)) // B) * B
    li = lax.broadcasted_iota(jnp.int32, (EPAD, EPAD), 0)
    lj = lax.broadcasted_iota(jnp.int32, (EPAD, EPAD), 1)
    slt = (li < lj).astype(jnp.bfloat16)
    start = lax.dot_general(
        pc.astype(jnp.bfloat16), slt, (((1,), (0,)), ((), ())),
        preferred_element_type=jnp.float32,
    ).astype(jnp.int32)

    start_b = jnp.broadcast_to(start.astype(jnp.float32), (T, EPAD))
    pos0 = jnp.sum(
        jnp.where(lane == idx1, start_b + rank0, 0.0), axis=1, keepdims=True
    )
    pos1 = jnp.sum(
        jnp.where(lane == idx2, start_b + rank1, 0.0), axis=1, keepdims=True
    )
    posc_ref[...] = jnp.where(
        lane == 0, pos0, jnp.where(lane == 1, pos1, 0.0)
    ).astype(jnp.int32)

    wexp_ref[0:T, :] = jnp.broadcast_to(m1, (T, EPAD))
    wexp_ref[T : 2 * T, :] = jnp.broadcast_to(m2, (T, EPAD))

    tstart = start // B
    tcnt = pc // B
    ti = lax.broadcasted_iota(jnp.int32, (1, EPAD), 1)
    texp = jnp.zeros((1, EPAD), jnp.int32)
    tval = jnp.zeros((1, EPAD), jnp.int32)
    for e in range(E):
        ts = tstart[0, e]
        tc = tcnt[0, e]
        mask = (ti >= ts) & (ti < ts + tc)
        texp = texp + e * mask.astype(jnp.int32)
        tval = tval + mask.astype(jnp.int32)
    texp_ref[...] = jnp.where(tval == 1, texp, E - 1)
    tval_ref[...] = tval


def _run_meta(xf, rw_pad):
    return pl.pallas_call(
        _meta_body,
        out_shape=(
            jax.ShapeDtypeStruct((T, EPAD), jnp.int32),
            jax.ShapeDtypeStruct((K * T, EPAD), jnp.float32),
            jax.ShapeDtypeStruct((1, EPAD), jnp.int32),
            jax.ShapeDtypeStruct((1, EPAD), jnp.int32),
        ),
        in_specs=[
            pl.BlockSpec((T, H), lambda: (0, 0)),
            pl.BlockSpec((EPAD, H), lambda: (0, 0)),
        ],
        out_specs=(
            pl.BlockSpec((T, EPAD), lambda: (0, 0)),
            pl.BlockSpec((K * T, EPAD), lambda: (0, 0)),
            pl.BlockSpec((1, EPAD), lambda: (0, 0)),
            pl.BlockSpec((1, EPAD), lambda: (0, 0)),
        ),
    )(xf, rw_pad)


def _disp_body(x_hbm, pos3_hbm, wexp_hbm, xs_hbm, gs_hbm,
               xv, wv0, wv1, idx0, idx1, sem0, sem1, sem2, sem3):
    wid = lax.axis_index("s") * NC + lax.axis_index("c")
    base = wid * CHUNK
    pltpu.sync_copy(x_hbm.at[pl.ds(base, CHUNK)], xv)
    pltpu.sync_copy(pos3_hbm.at[wid, 0], idx0)
    pltpu.sync_copy(pos3_hbm.at[wid, 1], idx1)
    pltpu.sync_copy(wexp_hbm.at[pl.ds(base, CHUNK)], wv0)
    pltpu.sync_copy(wexp_hbm.at[pl.ds(T + base, CHUNK)], wv1)
    c0 = pltpu.async_copy(xv, xs_hbm.at[idx0], sem0)
    c1 = pltpu.async_copy(xv, xs_hbm.at[idx1], sem1)
    c2 = pltpu.async_copy(wv0, gs_hbm.at[idx0], sem2)
    c3 = pltpu.async_copy(wv1, gs_hbm.at[idx1], sem3)
    c0.wait()
    c1.wait()
    c2.wait()
    c3.wait()


def _dispatch_sc(xf, pos3, wexp):
    run = functools.partial(
        pl.kernel,
        out_type=(
            jax.ShapeDtypeStruct((P, H), jnp.float32),
            jax.ShapeDtypeStruct((P, GW), jnp.float32),
        ),
        mesh=plsc.VectorSubcoreMesh(core_axis_name="c", subcore_axis_name="s"),
        scratch_types=[
            pltpu.VMEM((CHUNK, H), jnp.float32),
            pltpu.VMEM((CHUNK, GW), jnp.float32),
            pltpu.VMEM((CHUNK, GW), jnp.float32),
            pltpu.VMEM((CHUNK,), jnp.int32),
            pltpu.VMEM((CHUNK,), jnp.int32),
            pltpu.SemaphoreType.DMA,
            pltpu.SemaphoreType.DMA,
            pltpu.SemaphoreType.DMA,
            pltpu.SemaphoreType.DMA,
        ],
    )(_disp_body)
    return run(xf, pos3, wexp)


def _ffn_body(texp_ref, tval_ref, xs_ref, gs_ref, w1_ref, w2_ref,
              y_ref, xb_ref, sel_ref):
    c = pl.program_id(1)
    i = pl.program_id(0)

    @pl.when((i == 0) & (c == 0))
    def _():
        qi = lax.broadcasted_iota(jnp.int32, (2 * BI, BI), 0)
        ii = lax.broadcasted_iota(jnp.int32, (2 * BI, BI), 1)
        sel_ref[...] = (qi == 2 * ii + 1).astype(jnp.bfloat16)

    @pl.when(tval_ref[0, i] == 1)
    def _():
        @pl.when(c == 0)
        def _():
            xb_ref[...] = xs_ref[...].astype(jnp.bfloat16)

        x = xb_ref[...]
        wblk = w1_ref[0].astype(jnp.bfloat16)
        h = lax.dot_general(
            x, wblk, (((1,), (1,)), ((), ())),
            preferred_element_type=jnp.float32,
        )
        sh = h * jax.nn.sigmoid(h)
        shr = pltpu.roll(sh, 1, 1)
        p = (shr * h).astype(jnp.bfloat16)
        a = lax.dot_general(
            p, sel_ref[...], (((1,), (0,)), ((), ())),
            preferred_element_type=jnp.float32,
        ).astype(jnp.bfloat16)
        w2b = w2_ref[0].astype(jnp.bfloat16)
        partial = lax.dot_general(
            a, w2b, (((1,), (1,)), ((), ())),
            preferred_element_type=jnp.float32,
        )

        if NC_I == 1:
            y_ref[...] = partial * gs_ref[:, 0:1]
        else:

            @pl.when(c == 0)
            def _():
                y_ref[...] = partial

            @pl.when((c > 0) & (c < NC_I - 1))
            def _():
                y_ref[...] += partial

            @pl.when(c == NC_I - 1)
            def _():
                y_ref[...] = (y_ref[...] + partial) * gs_ref[:, 0:1]


def _run_ffn(texp, tval, xs, gs, w1p, w2):
    def _cidx(c, tval_ref, i):
        return jnp.where(tval_ref[0, i] == 1, c, 0)

    grid_spec = pltpu.PrefetchScalarGridSpec(
        num_scalar_prefetch=2,
        grid=(NT, NC_I),
        in_specs=[
            pl.BlockSpec((B, H), lambda i, c, texp, tval: (i, 0)),
            pl.BlockSpec((B, GW), lambda i, c, texp, tval: (i, 0)),
            pl.BlockSpec(
                (1, 2 * BI, H),
                lambda i, c, texp, tval: (texp[0, i], _cidx(c, tval, i), 0),
            ),
            pl.BlockSpec(
                (1, H, BI),
                lambda i, c, texp, tval: (texp[0, i], 0, _cidx(c, tval, i)),
            ),
        ],
        out_specs=pl.BlockSpec((B, H), lambda i, c, texp, tval: (i, 0)),
        scratch_shapes=[
            pltpu.VMEM((B, H), jnp.bfloat16),
            pltpu.VMEM((2 * BI, BI), jnp.bfloat16),
        ],
    )
    return pl.pallas_call(
        _ffn_body,
        grid_spec=grid_spec,
        out_shape=jax.ShapeDtypeStruct((P, H), jnp.float32),
        compiler_params=pltpu.CompilerParams(
            dimension_semantics=("arbitrary", "arbitrary"),
        ),
    )(texp, tval, xs, gs, w1p, w2)


def _comb_body(y_hbm, pos3_hbm, out_hbm, r0, r1, idx0, idx1, sem0, sem1):
    wid = lax.axis_index("s") * NC + lax.axis_index("c")
    base = wid * CHUNK
    for s in range(CHUNK // SUB):
        off = s * SUB
        pltpu.sync_copy(pos3_hbm.at[wid, 0, pl.ds(off, SUB)], idx0)
        pltpu.sync_copy(pos3_hbm.at[wid, 1, pl.ds(off, SUB)], idx1)
        c0 = pltpu.async_copy(y_hbm.at[idx0], r0, sem0)
        c1 = pltpu.async_copy(y_hbm.at[idx1], r1, sem1)
        c0.wait()
        c1.wait()

        def token_body(n, carry):
            def col_body(q4, carry2):
                for d in range(4):
                    sl = pl.ds(q4 * 64 + d * 16, 16)
                    r0[n, sl] = r0[n, sl] + r1[n, sl]
                return carry2

            return lax.fori_loop(0, H // 64, col_body, carry)

        lax.fori_loop(0, SUB, token_body, 0)
        pltpu.sync_copy(r0, out_hbm.at[pl.ds(base + off, SUB)])


def _combine_sc(y, pos3):
    run = functools.partial(
        pl.kernel,
        out_type=jax.ShapeDtypeStruct((T, H), jnp.float32),
        mesh=plsc.VectorSubcoreMesh(core_axis_name="c", subcore_axis_name="s"),
        scratch_types=[
            pltpu.VMEM((SUB, H), jnp.float32),
            pltpu.VMEM((SUB, H), jnp.float32),
            pltpu.VMEM((SUB,), jnp.int32),
            pltpu.VMEM((SUB,), jnp.int32),
            pltpu.SemaphoreType.DMA,
            pltpu.SemaphoreType.DMA,
        ],
    )(_comb_body)
    return run(y, pos3)


def kernel(hidden_states, router_w, w1, w2):
    orig_shape = hidden_states.shape
    xf = hidden_states.reshape(-1, orig_shape[-1])
    rw_pad = jnp.zeros((EPAD, H), jnp.float32).at[:E].set(router_w)

    posc, wexpc, texp, tval = _run_meta(xf, rw_pad)

    pos3 = jnp.stack(
        [posc[:, 0].reshape(NW, CHUNK), posc[:, 1].reshape(NW, CHUNK)], axis=1
    )
    wexp = wexpc

    xs, gs = _dispatch_sc(xf, pos3, wexp)
    y = _run_ffn(texp, tval, xs, gs, w1, w2)
    out = _combine_sc(y, pos3)
    return out.reshape(orig_shape)


# speedup vs baseline: 1.0602x; 1.0602x over previous
import functools

import jax
import jax.numpy as jnp
from jax import lax
from jax.experimental import pallas as pl
from jax.experimental.pallas import tpu as pltpu
from jax.experimental.pallas import tpu_sc as plsc

E = 8
K = 2
H = 1024
I = 2048
T = 2048
EPAD = 128

B = 256
NT = 23
P = NT * B
SELW = 512

NC = 2
NS = 16
NW = NC * NS
CHUNK = T // NW
SUB = 32
GW = 128


def _meta_body(x_ref, rw_ref, posc_ref, wexp_ref, texp_ref, tval_ref):
    x = x_ref[...]
    logits = lax.dot_general(
        x, rw_ref[...], (((1,), (1,)), ((), ())),
        preferred_element_type=jnp.float32,
    )
    lane = lax.broadcasted_iota(jnp.int32, (T, EPAD), 1)
    valid = lane < E
    logits = jnp.where(valid, logits, -1e30)
    m = jnp.max(logits, axis=1, keepdims=True)
    p = jnp.where(valid, jnp.exp(logits - m), 0.0)
    probs = p / jnp.sum(p, axis=1, keepdims=True)

    m1 = jnp.max(probs, axis=1, keepdims=True)
    idx1 = jnp.min(jnp.where(probs == m1, lane, EPAD), axis=1, keepdims=True)
    probs2 = jnp.where(lane == idx1, -1.0, probs)
    m2 = jnp.max(probs2, axis=1, keepdims=True)
    idx2 = jnp.min(jnp.where(probs2 == m2, lane, EPAD), axis=1, keepdims=True)

    oh0 = (lane == idx1).astype(jnp.bfloat16)
    oh1 = (lane == idx2).astype(jnp.bfloat16)

    r_i = lax.broadcasted_iota(jnp.int32, (T, T), 0)
    c_i = lax.broadcasted_iota(jnp.int32, (T, T), 1)
    s_tri = (r_i > c_i).astype(jnp.bfloat16)
    rank0 = lax.dot_general(
        s_tri, oh0, (((1,), (0,)), ((), ())),
        preferred_element_type=jnp.float32,
    )
    cnt0 = jnp.sum(oh0.astype(jnp.float32), axis=0, keepdims=True)
    rank1 = lax.dot_general(
        s_tri, oh1, (((1,), (0,)), ((), ())),
        preferred_element_type=jnp.float32,
    ) + cnt0
    cnt1 = jnp.sum(oh1.astype(jnp.float32), axis=0, keepdims=True)
    counts = (cnt0 + cnt1).astype(jnp.int32)

    pc = ((counts + (B - 1)) // B) * B
    li = lax.broadcasted_iota(jnp.int32, (EPAD, EPAD), 0)
    lj = lax.broadcasted_iota(jnp.int32, (EPAD, EPAD), 1)
    slt = (li < lj).astype(jnp.bfloat16)
    start = lax.dot_general(
        pc.astype(jnp.bfloat16), slt, (((1,), (0,)), ((), ())),
        preferred_element_type=jnp.float32,
    ).astype(jnp.int32)

    start_b = jnp.broadcast_to(start.astype(jnp.float32), (T, EPAD))
    pos0 = jnp.sum(
        jnp.where(lane == idx1, start_b + rank0, 0.0), axis=1, keepdims=True
    )
    pos1 = jnp.sum(
        jnp.where(lane == idx2, start_b + rank1, 0.0), axis=1, keepdims=True
    )
    posc_ref[...] = jnp.where(
        lane == 0, pos0, jnp.where(lane == 1, pos1, 0.0)
    ).astype(jnp.int32)

    wexp_ref[0:T, :] = jnp.broadcast_to(m1, (T, EPAD))
    wexp_ref[T : 2 * T, :] = jnp.broadcast_to(m2, (T, EPAD))

    tstart = start // B
    tcnt = pc // B
    ti = lax.broadcasted_iota(jnp.int32, (1, EPAD), 1)
    texp = jnp.zeros((1, EPAD), jnp.int32)
    tval = jnp.zeros((1, EPAD), jnp.int32)
    for e in range(E):
        ts = tstart[0, e]
        tc = tcnt[0, e]
        mask = (ti >= ts) & (ti < ts + tc)
        texp = texp + e * mask.astype(jnp.int32)
        tval = tval + mask.astype(jnp.int32)
    last_e = jnp.max(texp * tval)
    texp_ref[...] = jnp.where(tval == 1, texp, last_e)
    tval_ref[...] = tval


def _run_meta(xf, rw_pad):
    return pl.pallas_call(
        _meta_body,
        out_shape=(
            jax.ShapeDtypeStruct((T, EPAD), jnp.int32),
            jax.ShapeDtypeStruct((K * T, EPAD), jnp.float32),
            jax.ShapeDtypeStruct((1, EPAD), jnp.int32),
            jax.ShapeDtypeStruct((1, EPAD), jnp.int32),
        ),
        in_specs=[
            pl.BlockSpec((T, H), lambda: (0, 0)),
            pl.BlockSpec((EPAD, H), lambda: (0, 0)),
        ],
        out_specs=(
            pl.BlockSpec((T, EPAD), lambda: (0, 0)),
            pl.BlockSpec((K * T, EPAD), lambda: (0, 0)),
            pl.BlockSpec((1, EPAD), lambda: (0, 0)),
            pl.BlockSpec((1, EPAD), lambda: (0, 0)),
        ),
    )(xf, rw_pad)


def _disp_body(x_hbm, pos3_hbm, wexp_hbm, xs_hbm, gs_hbm,
               xv, wv0, wv1, idx0, idx1, sem0, sem1, sem2, sem3):
    wid = lax.axis_index("s") * NC + lax.axis_index("c")
    base = wid * CHUNK
    pltpu.sync_copy(x_hbm.at[pl.ds(base, CHUNK)], xv)
    pltpu.sync_copy(pos3_hbm.at[wid, 0], idx0)
    pltpu.sync_copy(pos3_hbm.at[wid, 1], idx1)
    pltpu.sync_copy(wexp_hbm.at[pl.ds(base, CHUNK)], wv0)
    pltpu.sync_copy(wexp_hbm.at[pl.ds(T + base, CHUNK)], wv1)
    c0 = pltpu.async_copy(xv, xs_hbm.at[idx0], sem0)
    c1 = pltpu.async_copy(xv, xs_hbm.at[idx1], sem1)
    c2 = pltpu.async_copy(wv0, gs_hbm.at[idx0], sem2)
    c3 = pltpu.async_copy(wv1, gs_hbm.at[idx1], sem3)
    c0.wait()
    c1.wait()
    c2.wait()
    c3.wait()


def _dispatch_sc(xf, pos3, wexp):
    run = functools.partial(
        pl.kernel,
        out_type=(
            jax.ShapeDtypeStruct((P, H), jnp.float32),
            jax.ShapeDtypeStruct((P, GW), jnp.float32),
        ),
        mesh=plsc.VectorSubcoreMesh(core_axis_name="c", subcore_axis_name="s"),
        scratch_types=[
            pltpu.VMEM((CHUNK, H), jnp.float32),
            pltpu.VMEM((CHUNK, GW), jnp.float32),
            pltpu.VMEM((CHUNK, GW), jnp.float32),
            pltpu.VMEM((CHUNK,), jnp.int32),
            pltpu.VMEM((CHUNK,), jnp.int32),
            pltpu.SemaphoreType.DMA,
            pltpu.SemaphoreType.DMA,
            pltpu.SemaphoreType.DMA,
            pltpu.SemaphoreType.DMA,
        ],
    )(_disp_body)
    return run(xf, pos3, wexp)


def _ffn_body(texp_ref, tval_ref, xs_ref, gs_ref, w1_ref, w2_ref,
              y_ref, sel_ref):
    i = pl.program_id(0)

    @pl.when(i == 0)
    def _():
        qi = lax.broadcasted_iota(jnp.int32, (2 * SELW, SELW), 0)
        ii = lax.broadcasted_iota(jnp.int32, (2 * SELW, SELW), 1)
        sel_ref[...] = (qi == 2 * ii + 1).astype(jnp.bfloat16)

    @pl.when(tval_ref[0, i] == 1)
    def _():
        x = xs_ref[...].astype(jnp.bfloat16)
        wblk = w1_ref[0].astype(jnp.bfloat16)
        h = lax.dot_general(
            x, wblk, (((1,), (1,)), ((), ())),
            preferred_element_type=jnp.float32,
        )
        sh = h * jax.nn.sigmoid(h)
        shr = pltpu.roll(sh, 1, 1)
        p = (shr * h).astype(jnp.bfloat16)
        sel = sel_ref[...]
        parts = [
            lax.dot_general(
                p[:, 2 * SELW * k : 2 * SELW * (k + 1)], sel,
                (((1,), (0,)), ((), ())),
                preferred_element_type=jnp.float32,
            )
            for k in range(I // SELW)
        ]
        a = jnp.concatenate(parts, axis=1).astype(jnp.bfloat16)
        w2b = w2_ref[0].astype(jnp.bfloat16)
        partial = lax.dot_general(
            a, w2b, (((1,), (1,)), ((), ())),
            preferred_element_type=jnp.float32,
        )
        y_ref[...] = partial * gs_ref[:, 0:1]


def _run_ffn(texp, tval, xs, gs, w1p, w2):
    grid_spec = pltpu.PrefetchScalarGridSpec(
        num_scalar_prefetch=2,
        grid=(NT,),
        in_specs=[
            pl.BlockSpec((B, H), lambda i, texp, tval: (i, 0)),
            pl.BlockSpec((B, GW), lambda i, texp, tval: (i, 0)),
            pl.BlockSpec(
                (1, 2 * I, H), lambda i, texp, tval: (texp[0, i], 0, 0)
            ),
            pl.BlockSpec(
                (1, H, I), lambda i, texp, tval: (texp[0, i], 0, 0)
            ),
        ],
        out_specs=pl.BlockSpec((B, H), lambda i, texp, tval: (i, 0)),
        scratch_shapes=[
            pltpu.VMEM((2 * SELW, SELW), jnp.bfloat16),
        ],
    )
    return pl.pallas_call(
        _ffn_body,
        grid_spec=grid_spec,
        out_shape=jax.ShapeDtypeStruct((P, H), jnp.float32),
        compiler_params=pltpu.CompilerParams(
            dimension_semantics=("arbitrary",),
            vmem_limit_bytes=110 * 1024 * 1024,
        ),
    )(texp, tval, xs, gs, w1p, w2)


def _comb_body(y_hbm, pos3_hbm, out_hbm, r0, r1, idx0, idx1, sem0, sem1):
    wid = lax.axis_index("s") * NC + lax.axis_index("c")
    base = wid * CHUNK
    for s in range(CHUNK // SUB):
        off = s * SUB
        pltpu.sync_copy(pos3_hbm.at[wid, 0, pl.ds(off, SUB)], idx0)
        pltpu.sync_copy(pos3_hbm.at[wid, 1, pl.ds(off, SUB)], idx1)
        c0 = pltpu.async_copy(y_hbm.at[idx0], r0, sem0)
        c1 = pltpu.async_copy(y_hbm.at[idx1], r1, sem1)
        c0.wait()
        c1.wait()

        def token_body(n, carry):
            def col_body(q4, carry2):
                for d in range(4):
                    sl = pl.ds(q4 * 64 + d * 16, 16)
                    r0[n, sl] = r0[n, sl] + r1[n, sl]
                return carry2

            return lax.fori_loop(0, H // 64, col_body, carry)

        lax.fori_loop(0, SUB, token_body, 0)
        pltpu.sync_copy(r0, out_hbm.at[pl.ds(base + off, SUB)])


def _combine_sc(y, pos3):
    run = functools.partial(
        pl.kernel,
        out_type=jax.ShapeDtypeStruct((T, H), jnp.float32),
        mesh=plsc.VectorSubcoreMesh(core_axis_name="c", subcore_axis_name="s"),
        scratch_types=[
            pltpu.VMEM((SUB, H), jnp.float32),
            pltpu.VMEM((SUB, H), jnp.float32),
            pltpu.VMEM((SUB,), jnp.int32),
            pltpu.VMEM((SUB,), jnp.int32),
            pltpu.SemaphoreType.DMA,
            pltpu.SemaphoreType.DMA,
        ],
    )(_comb_body)
    return run(y, pos3)


def kernel(hidden_states, router_w, w1, w2):
    orig_shape = hidden_states.shape
    xf = hidden_states.reshape(-1, orig_shape[-1])
    rw_pad = jnp.zeros((EPAD, H), jnp.float32).at[:E].set(router_w)

    posc, wexpc, texp, tval = _run_meta(xf, rw_pad)

    pos3 = jnp.stack(
        [posc[:, 0].reshape(NW, CHUNK), posc[:, 1].reshape(NW, CHUNK)], axis=1
    )
    wexp = wexpc

    xs, gs = _dispatch_sc(xf, pos3, wexp)
    y = _run_ffn(texp, tval, xs, gs, w1, w2)
    out = _combine_sc(y, pos3)
    return out.reshape(orig_shape)
